# Initial kernel scaffold; baseline (speedup 1.0000x reference)
import jax
import jax.numpy as jnp
from jax.experimental import pallas as pl


def kernel(xyz, new_xyz, features):
    raise NotImplementedError("write your pallas kernel here")


# speedup vs baseline: 6.3618x; 6.3618x over previous
import jax
import jax.numpy as jnp
from jax import lax
from jax.experimental import pallas as pl

_R2 = 0.2 * 0.2
_NS = 32
_MB = 128
_NC = 512
_CPAD = 72


def _qg_body(qT_ref, xyzn_ref, fx_ref, out_ref):
    fx = fx_ref[0]
    xn = xyzn_ref[0]
    qT = qT_ref[0]
    n_total = fx.shape[1]

    ii = lax.broadcasted_iota(jnp.int32, (_NC, _NC), 0)
    jj = lax.broadcasted_iota(jnp.int32, (_NC, _NC), 1)
    ltri = (ii >= jj).astype(jnp.bfloat16)

    qx = qT[0:1, :]
    qy = qT[1:2, :]
    qz = qT[2:3, :]
    sqq = (qx * qx + qy * qy) + qz * qz
    qxb = qx.astype(jnp.bfloat16).astype(jnp.float32)
    qyb = qy.astype(jnp.bfloat16).astype(jnp.float32)
    qzb = qz.astype(jnp.bfloat16).astype(jnp.float32)

    acc = jnp.zeros((_CPAD, _NS * _MB), jnp.float32)
    k0 = jnp.zeros((1, _MB), jnp.float32)
    for ci in range(n_total // _NC):
        sl = slice(ci * _NC, (ci + 1) * _NC)
        px = xn[sl, 0:1]
        py = xn[sl, 1:2]
        pz = xn[sl, 2:3]
        sqx = (px * px + py * py) + pz * pz
        pxb = px.astype(jnp.bfloat16).astype(jnp.float32)
        pyb = py.astype(jnp.bfloat16).astype(jnp.float32)
        pzb = pz.astype(jnp.bfloat16).astype(jnp.float32)
        qp = (pxb * qxb + pyb * qyb) + pzb * qzb
        dist = (sqq + sqx) - 2.0 * qp
        mk = dist < _R2
        mkb = jnp.where(mk, 1.0, 0.0).astype(jnp.bfloat16)
        c = jnp.dot(ltri, mkb, preferred_element_type=jnp.float32)
        s = jnp.where(mk, c + k0, 0.0)
        es = [(s == float(k)).astype(jnp.bfloat16) for k in range(1, _NS + 1)]
        e_all = jnp.concatenate(es, axis=1)
        fxc = fx[:, sl].astype(jnp.bfloat16)
        acc = acc + jnp.dot(fxc, e_all, preferred_element_type=jnp.float32)
        k0 = k0 + c[_NC - 1:_NC, :]

    slot1 = acc[:, 0:_MB]
    feat0 = fx[:, 0:1]
    fill = jnp.where(k0 > 0.5, slot1, feat0)
    for k in range(1, _NS + 1):
        col = acc[:, (k - 1) * _MB:k * _MB]
        col = jnp.where(k0 >= float(k), col, fill)
        xyzp = col[0:3, :] - qT
        out_ref[0, :, :, k - 1] = jnp.concatenate([xyzp, col[3:67, :]], axis=0)


def kernel(xyz, new_xyz, features):
    B, N, _ = xyz.shape
    M = new_xyz.shape[1]
    C = features.shape[1]
    xyz_t = jnp.transpose(xyz, (0, 2, 1))
    pad = _CPAD - 3 - C
    featx = jnp.concatenate(
        [xyz_t, features, jnp.zeros((B, pad, N), jnp.float32)], axis=1)
    qT = jnp.transpose(new_xyz, (0, 2, 1))
    return pl.pallas_call(
        _qg_body,
        grid=(B, M // _MB),
        in_specs=[
            pl.BlockSpec((1, 3, _MB), lambda b, mb: (b, 0, mb)),
            pl.BlockSpec((1, N, 3), lambda b, mb: (b, 0, 0)),
            pl.BlockSpec((1, _CPAD, N), lambda b, mb: (b, 0, 0)),
        ],
        out_specs=pl.BlockSpec((1, 3 + C, _MB, _NS), lambda b, mb: (b, 0, mb, 0)),
        out_shape=jax.ShapeDtypeStruct((B, 3 + C, M, _NS), jnp.float32),
    )(qT, xyz, featx)


# speedup vs baseline: 21.6126x; 3.3973x over previous
import jax
import jax.numpy as jnp
from jax import lax
from jax.experimental import pallas as pl

_R2 = 0.2 * 0.2
_NS = 32
_MB = 128
_NC = 512
_CPAD = 72


def _qg_body(qT_ref, xyzn_ref, fx_ref, out_ref):
    fx = fx_ref[0]
    xn = xyzn_ref[0]
    qT = qT_ref[0]
    n_total = fx.shape[1]

    ii = lax.broadcasted_iota(jnp.int32, (_NC, _NC), 0)
    jj = lax.broadcasted_iota(jnp.int32, (_NC, _NC), 1)
    ltri = (ii >= jj).astype(jnp.bfloat16)

    qx = qT[0:1, :]
    qy = qT[1:2, :]
    qz = qT[2:3, :]
    sqq = (qx * qx + qy * qy) + qz * qz
    qxb = qx.astype(jnp.bfloat16).astype(jnp.float32)
    qyb = qy.astype(jnp.bfloat16).astype(jnp.float32)
    qzb = qz.astype(jnp.bfloat16).astype(jnp.float32)

    acc = jnp.zeros((_CPAD, _NS * _MB), jnp.float32)
    k0 = jnp.zeros((1, _MB), jnp.float32)
    for ci in range(n_total // _NC):
        sl = slice(ci * _NC, (ci + 1) * _NC)
        px = xn[sl, 0:1]
        py = xn[sl, 1:2]
        pz = xn[sl, 2:3]
        sqx = (px * px + py * py) + pz * pz
        pxb = px.astype(jnp.bfloat16).astype(jnp.float32)
        pyb = py.astype(jnp.bfloat16).astype(jnp.float32)
        pzb = pz.astype(jnp.bfloat16).astype(jnp.float32)
        qp = (pxb * qxb + pyb * qyb) + pzb * qzb
        dist = (sqq + sqx) - 2.0 * qp
        mk = dist < _R2
        mkb = jnp.where(mk, 1.0, 0.0).astype(jnp.bfloat16)
        c = jnp.dot(ltri, mkb, preferred_element_type=jnp.float32)
        s = jnp.where(mk, c + k0, 0.0)
        es = [(s == float(k)).astype(jnp.bfloat16) for k in range(1, _NS + 1)]
        e_all = jnp.concatenate(es, axis=1)
        fxc = fx[:, sl].astype(jnp.bfloat16)
        acc = acc + jnp.dot(fxc, e_all, preferred_element_type=jnp.float32)
        k0 = k0 + c[_NC - 1:_NC, :]

    slot1 = acc[:, 0:_MB]
    feat0 = fx[:, 0:1]
    fill = jnp.where(k0 > 0.5, slot1, feat0)
    for k in range(1, _NS + 1):
        col = acc[:, (k - 1) * _MB:k * _MB]
        col = jnp.where(k0 >= float(k), col, fill)
        xyzp = col[0:3, :] - qT
        out_ref[0, :, k - 1, :] = jnp.concatenate([xyzp, col[3:67, :]], axis=0)


def kernel(xyz, new_xyz, features):
    B, N, _ = xyz.shape
    M = new_xyz.shape[1]
    C = features.shape[1]
    xyz_t = jnp.transpose(xyz, (0, 2, 1))
    pad = _CPAD - 3 - C
    featx = jnp.concatenate(
        [xyz_t, features, jnp.zeros((B, pad, N), jnp.float32)], axis=1)
    qT = jnp.transpose(new_xyz, (0, 2, 1))
    return pl.pallas_call(
        _qg_body,
        grid=(B, M // _MB),
        in_specs=[
            pl.BlockSpec((1, 3, _MB), lambda b, mb: (b, 0, mb)),
            pl.BlockSpec((1, N, 3), lambda b, mb: (b, 0, 0)),
            pl.BlockSpec((1, _CPAD, N), lambda b, mb: (b, 0, 0)),
        ],
        out_specs=pl.BlockSpec((1, 3 + C, _NS, _MB), lambda b, mb: (b, 0, 0, mb)),
        out_shape=jax.ShapeDtypeStruct((B, 3 + C, _NS, M), jnp.float32),
    )(qT, xyz, featx).transpose(0, 1, 3, 2)


# speedup vs baseline: 25.6843x; 1.1884x over previous
import jax
import jax.numpy as jnp
from jax import lax
from jax.experimental import pallas as pl

_R2 = 0.2 * 0.2
_NS = 32
_MB = 128
_NC = 512
_CPAD = 72


def _qg_body(qT_ref, xyzn_ref, fx_ref, out_ref):
    fx = fx_ref[0]
    xn = xyzn_ref[0]
    qT = qT_ref[0]
    n_total = fx.shape[1]

    ii = lax.broadcasted_iota(jnp.int32, (_NC, _NC), 0)
    jj = lax.broadcasted_iota(jnp.int32, (_NC, _NC), 1)
    ltri = (ii >= jj).astype(jnp.float8_e4m3fn)
    fx_hi = fx.astype(jnp.float8_e4m3fn)
    fx_lo = (fx - fx_hi.astype(jnp.float32)).astype(jnp.float8_e4m3fn)
    fx8 = jnp.concatenate([fx_hi, fx_lo], axis=0)

    qx = qT[0:1, :]
    qy = qT[1:2, :]
    qz = qT[2:3, :]
    sqq = (qx * qx + qy * qy) + qz * qz
    qxb = qx.astype(jnp.bfloat16).astype(jnp.float32)
    qyb = qy.astype(jnp.bfloat16).astype(jnp.float32)
    qzb = qz.astype(jnp.bfloat16).astype(jnp.float32)

    acc2 = jnp.zeros((2 * _CPAD, _NS * _MB), jnp.float32)
    k0 = jnp.zeros((1, _MB), jnp.float32)
    for ci in range(n_total // _NC):
        sl = slice(ci * _NC, (ci + 1) * _NC)
        px = xn[sl, 0:1]
        py = xn[sl, 1:2]
        pz = xn[sl, 2:3]
        sqx = (px * px + py * py) + pz * pz
        pxb = px.astype(jnp.bfloat16).astype(jnp.float32)
        pyb = py.astype(jnp.bfloat16).astype(jnp.float32)
        pzb = pz.astype(jnp.bfloat16).astype(jnp.float32)
        qp = (pxb * qxb + pyb * qyb) + pzb * qzb
        dist = (sqq + sqx) - 2.0 * qp
        mk = dist < _R2
        mkb = jnp.where(mk, 1.0, 0.0).astype(jnp.float8_e4m3fn)
        c = jnp.dot(ltri, mkb, preferred_element_type=jnp.float32)
        s = jnp.where(mk, c + k0, 0.0)
        s_bf = jnp.minimum(s, 48.0).astype(jnp.bfloat16)
        one8 = jnp.bfloat16(1)
        zero8 = jnp.bfloat16(0)
        es = [jnp.where(s_bf == jnp.bfloat16(k), one8,
                        zero8).astype(jnp.float8_e4m3fn)
              for k in range(1, _NS + 1)]
        e_all = jnp.concatenate(es, axis=1)
        acc2 = acc2 + jnp.dot(fx8[:, sl], e_all,
                              preferred_element_type=jnp.float32)
        k0 = k0 + c[_NC - 1:_NC, :]

    acc = acc2[0:_CPAD, :] + acc2[_CPAD:2 * _CPAD, :]

    slot1 = acc[:, 0:_MB]
    feat0 = fx[:, 0:1]
    fill = jnp.where(k0 > 0.5, slot1, feat0)
    for k in range(1, _NS + 1):
        col = acc[:, (k - 1) * _MB:k * _MB]
        col = jnp.where(k0 >= float(k), col, fill)
        xyzp = col[0:3, :] - qT
        out_ref[0, :, k - 1, :] = jnp.concatenate([xyzp, col[3:67, :]], axis=0)


def kernel(xyz, new_xyz, features):
    B, N, _ = xyz.shape
    M = new_xyz.shape[1]
    C = features.shape[1]
    xyz_t = jnp.transpose(xyz, (0, 2, 1))
    pad = _CPAD - 3 - C
    featx = jnp.concatenate(
        [xyz_t, features, jnp.zeros((B, pad, N), jnp.float32)], axis=1)
    qT = jnp.transpose(new_xyz, (0, 2, 1))
    return pl.pallas_call(
        _qg_body,
        grid=(B, M // _MB),
        in_specs=[
            pl.BlockSpec((1, 3, _MB), lambda b, mb: (b, 0, mb)),
            pl.BlockSpec((1, N, 3), lambda b, mb: (b, 0, 0)),
            pl.BlockSpec((1, _CPAD, N), lambda b, mb: (b, 0, 0)),
        ],
        out_specs=pl.BlockSpec((1, 3 + C, _NS, _MB), lambda b, mb: (b, 0, 0, mb)),
        out_shape=jax.ShapeDtypeStruct((B, 3 + C, _NS, M), jnp.float32),
    )(qT, xyz, featx).transpose(0, 1, 3, 2)
